# baseline (speedup 1.0000x reference)
import functools

import jax
import jax.numpy as jnp
from jax import lax
from jax.experimental import pallas as pl
from jax.experimental.pallas import tpu as pltpu
from jax.experimental.pallas import tpu_sc as plsc

_NC = 2
_NS = 16
_NW = _NC * _NS

_GROW = 100
_RPG = 4


def _ln_t_body(tab_ref, g_ref, b_ref, out_ref):
    x = tab_ref[...]
    mean = jnp.mean(x, axis=0, keepdims=True)
    c = x - mean
    var = jnp.mean(c * c, axis=0, keepdims=True)
    out_ref[...] = c * lax.rsqrt(var + 1e-5) * g_ref[...] + b_ref[...]


def _normalize_table_t(table_t, gamma, beta):
    d, v = table_t.shape
    blk = 16384
    return pl.pallas_call(
        _ln_t_body,
        grid=(pl.cdiv(v, blk),),
        in_specs=[
            pl.BlockSpec((d, blk), lambda i: (0, i)),
            pl.BlockSpec((d, 1), lambda i: (0, 0)),
            pl.BlockSpec((d, 1), lambda i: (0, 0)),
        ],
        out_specs=pl.BlockSpec((d, blk), lambda i: (0, i)),
        out_shape=jax.ShapeDtypeStruct((d, v), jnp.float32),
    )(table_t, gamma.reshape(d, 1), beta.reshape(d, 1))


def _sc_transpose(tab_t):
    d, v = tab_t.shape
    ch = 800
    n_chunks = v // ch
    base_per_w = n_chunks // _NW
    extra_ws = n_chunks - base_per_w * _NW

    mesh = plsc.VectorSubcoreMesh(core_axis_name="c", subcore_axis_name="s")

    @functools.partial(
        pl.kernel,
        mesh=mesh,
        out_type=jax.ShapeDtypeStruct((v, d), jnp.float32),
        compiler_params=pltpu.CompilerParams(
            use_tc_tiling_on_sc=False, needs_layout_passes=False
        ),
        scratch_types=[
            pltpu.VMEM((2, d, ch), jnp.float32),
            pltpu.VMEM((2, ch, d), jnp.float32),
            pltpu.SemaphoreType.DMA,
            pltpu.SemaphoreType.DMA,
        ],
    )
    def t(tab_t_hbm, out_hbm, in_v, out_v, sem0, sem1):
        wid = lax.axis_index("s") * _NC + lax.axis_index("c")
        n_mine = base_per_w + jnp.where(wid < extra_ws, 1, 0)
        sems = (sem0, sem1)

        def in_copy(b, i):
            return pltpu.make_async_copy(
                tab_t_hbm.at[:, pl.ds((wid + i * _NW) * ch, ch)], in_v.at[b], sems[b]
            )

        def transpose(b):
            iota = lax.iota(jnp.int32, 16)

            def col_grp(g, carry):
                ridx = g * 16 + iota
                for e in range(d):
                    val = in_v[b, e, pl.ds(g * 16, 16)]
                    plsc.store_scatter(
                        out_v.at[b], [ridx, jnp.full((16,), e, jnp.int32)], val
                    )
                return carry

            lax.fori_loop(0, ch // 16, col_grp, None)

        def store(b, i):
            pltpu.sync_copy(out_v.at[b], out_hbm.at[pl.ds((wid + i * _NW) * ch, ch)])

        in_copy(0, 0).start()

        def step(i, carry):
            b = lax.rem(i, 2)

            @pl.when(i + 1 < n_mine)
            def _():
                @pl.when(b == 0)
                def _():
                    in_copy(1, i + 1).start()

                @pl.when(b == 1)
                def _():
                    in_copy(0, i + 1).start()

            @pl.when(b == 0)
            def _():
                in_copy(0, i).wait()
                transpose(0)
                store(0, i)

            @pl.when(b == 1)
            def _():
                in_copy(1, i).wait()
                transpose(1)
                store(1, i)

            return carry

        lax.fori_loop(0, n_mine, step, None)

    return t(tab_t)


def _sc_gather(tab, idx2d, batch, hist):
    d = tab.shape[1]
    rows_per_w = batch // _NW
    n_groups = rows_per_w // _RPG
    n_pairs = n_groups // 2
    dpr = hist // _GROW
    dpg = _RPG * dpr
    ipg = _RPG * hist // _GROW

    mesh = plsc.VectorSubcoreMesh(core_axis_name="c", subcore_axis_name="s")

    @functools.partial(
        pl.kernel,
        mesh=mesh,
        out_type=jax.ShapeDtypeStruct((batch, hist, 128), jnp.float32),
        compiler_params=pltpu.CompilerParams(use_tc_tiling_on_sc=False),
        scratch_types=[
            pltpu.VMEM((2, dpg, _GROW), jnp.int32),
            pltpu.VMEM((2, _RPG, hist, d), jnp.float32),
            pltpu.SemaphoreType.DMA,
            pltpu.SemaphoreType.DMA,
        ],
    )
    def k(tab_hbm, idx_hbm, out_hbm, idx_v, rows_v, sem0, sem1):
        wid = lax.axis_index("s") * _NC + lax.axis_index("c")
        ibase = wid * n_groups * ipg
        obase = wid * rows_per_w
        sems = (sem0, sem1)

        def load_idx(b, g):
            pltpu.sync_copy(idx_hbm.at[pl.ds(ibase + g * ipg, ipg)], idx_v.at[b])

        def descs(b):
            for j in range(dpg):
                yield (
                    tab_hbm.at[idx_v.at[b, j]],
                    rows_v.at[b, j // dpr, pl.ds((j % dpr) * _GROW, _GROW)],
                    sems[b],
                )

        def fire(b):
            for src, dst, sem in descs(b):
                pltpu.make_async_copy(src, dst, sem).start()

        def drain(b):
            for src, dst, sem in descs(b):
                pltpu.make_async_copy(src, dst, sem).wait()

        def store(b, g):
            pltpu.sync_copy(
                rows_v.at[b],
                out_hbm.at[pl.ds(obase + g * _RPG, _RPG), :, pl.ds(0, d)],
            )

        load_idx(0, 0)
        fire(0)

        def pair(i, carry):
            g_a = 2 * i
            g_b = g_a + 1
            load_idx(1, g_b)
            fire(1)
            drain(0)
            store(0, g_a)

            @pl.when(i + 1 < n_pairs)
            def _():
                load_idx(0, g_a + 2)
                fire(0)

            drain(1)
            store(1, g_b)
            return carry

        lax.fori_loop(0, n_pairs, pair, None)

    return k(tab, idx2d)


def kernel(indices, table, gamma, beta):
    batch, hist = indices.shape
    d = table.shape[1]
    norm_t = _normalize_table_t(table.T, gamma, beta)
    idx2d = indices.astype(jnp.int32).reshape(batch * hist // _GROW, _GROW)
    norm_rows = _sc_transpose(norm_t)
    padded = _sc_gather(norm_rows, idx2d, batch, hist)
    return padded[:, :, :d]


# speedup vs baseline: 3.4164x; 3.4164x over previous
import functools

import jax
import jax.numpy as jnp
from jax import lax
from jax.experimental import pallas as pl
from jax.experimental.pallas import tpu as pltpu
from jax.experimental.pallas import tpu_sc as plsc

_NC = 2
_NS = 16
_NW = _NC * _NS

_GROW = 100
_RPG = 4


def _ln_t_body(tab_ref, g_ref, b_ref, out_ref):
    x = tab_ref[...]
    mean = jnp.mean(x, axis=0, keepdims=True)
    c = x - mean
    var = jnp.mean(c * c, axis=0, keepdims=True)
    xn = c * lax.rsqrt(var + 1e-5) * g_ref[...] + b_ref[...]
    bn = xn.shape[1]
    q = bn // 4
    xnt = xn.T
    out_ref[...] = jnp.concatenate(
        [xnt[k * q:(k + 1) * q, :] for k in range(4)], axis=1
    )


def _normalize_table_t(table_t, gamma, beta):
    d, v = table_t.shape
    blk = 16384
    return pl.pallas_call(
        _ln_t_body,
        grid=(pl.cdiv(v, blk),),
        in_specs=[
            pl.BlockSpec((d, blk), lambda i: (0, i)),
            pl.BlockSpec((d, 1), lambda i: (0, 0)),
            pl.BlockSpec((d, 1), lambda i: (0, 0)),
        ],
        out_specs=pl.BlockSpec((blk // 4, 128), lambda i: (i, 0)),
        out_shape=jax.ShapeDtypeStruct(
            (pl.cdiv(v, blk) * (blk // 4), 128), jnp.float32
        ),
    )(table_t, gamma.reshape(d, 1), beta.reshape(d, 1))


def _sc_gather(tab, idx2d, batch, hist):
    d = tab.shape[1]
    rows_per_w = batch // _NW
    n_groups = rows_per_w // _RPG
    n_pairs = n_groups // 2
    dpr = hist // _GROW
    dpg = _RPG * dpr
    ipg = _RPG * hist // _GROW

    mesh = plsc.VectorSubcoreMesh(core_axis_name="c", subcore_axis_name="s")

    @functools.partial(
        pl.kernel,
        mesh=mesh,
        out_type=jax.ShapeDtypeStruct((batch, hist, 128), jnp.float32),
        compiler_params=pltpu.CompilerParams(use_tc_tiling_on_sc=False),
        scratch_types=[
            pltpu.VMEM((2, dpg, _GROW), jnp.int32),
            pltpu.VMEM((2, _RPG, hist, d), jnp.float32),
            pltpu.SemaphoreType.DMA,
            pltpu.SemaphoreType.DMA,
        ],
    )
    def k(tab_hbm, idx_hbm, out_hbm, idx_v, rows_v, sem0, sem1):
        wid = lax.axis_index("s") * _NC + lax.axis_index("c")
        ibase = wid * n_groups * ipg
        obase = wid * rows_per_w
        sems = (sem0, sem1)

        def load_idx(b, g):
            pltpu.sync_copy(idx_hbm.at[pl.ds(ibase + g * ipg, ipg)], idx_v.at[b])

        def descs(b):
            for j in range(dpg):
                yield (
                    tab_hbm.at[idx_v.at[b, j]],
                    rows_v.at[b, j // dpr, pl.ds((j % dpr) * _GROW, _GROW)],
                    sems[b],
                )

        def fire(b):
            for src, dst, sem in descs(b):
                pltpu.make_async_copy(src, dst, sem).start()

        def drain(b):
            for src, dst, sem in descs(b):
                pltpu.make_async_copy(src, dst, sem).wait()

        def store(b, g):
            pltpu.sync_copy(
                rows_v.at[b],
                out_hbm.at[pl.ds(obase + g * _RPG, _RPG), :, pl.ds(0, d)],
            )

        load_idx(0, 0)
        fire(0)

        def pair(i, carry):
            g_a = 2 * i
            g_b = g_a + 1
            load_idx(1, g_b)
            fire(1)
            drain(0)
            store(0, g_a)

            @pl.when(i + 1 < n_pairs)
            def _():
                load_idx(0, g_a + 2)
                fire(0)

            drain(1)
            store(1, g_b)
            return carry

        lax.fori_loop(0, n_pairs, pair, None)

    return k(tab, idx2d)


def kernel(indices, table, gamma, beta):
    batch, hist = indices.shape
    d = table.shape[1]
    v = table.shape[0]
    norm_packed = _normalize_table_t(table.T, gamma, beta)
    norm_rows = norm_packed.reshape(norm_packed.shape[0] * 128 // d, d)
    idx = indices.astype(jnp.int32)
    idx = (idx & ~16383) | ((idx & 4095) << 2) | ((idx >> 12) & 3)
    idx2d = idx.reshape(batch * hist // _GROW, _GROW)
    padded = _sc_gather(norm_rows, idx2d, batch, hist)
    return padded[:, :, :d]
